# Initial kernel scaffold; baseline (speedup 1.0000x reference)
import jax
import jax.numpy as jnp
from jax.experimental import pallas as pl


def kernel(params, sub, rel, edge_index, edge_type):
    raise NotImplementedError("write your pallas kernel here")


# speedup vs baseline: 1.8591x; 1.8591x over previous
import functools
import numpy as np
import jax
import jax.numpy as jnp
from jax.experimental import pallas as pl

NUM_ENT = 10000
NUM_REL = 500
E = 160000
INIT_DIM = 128
EMBED_DIM = 128
HEADS = 2
HIDDEN = EMBED_DIM // HEADS
K_W, K_H = 8, 16
KER = 7
NFILT = 96
FLAT_SZ = (2 * K_W - KER + 1) * (K_H - KER + 1) * NFILT
BATCH = 1024
N_TOTAL = NUM_ENT + E
ALPHA = 0.2


def _mm_body(x_ref, w_ref, o_ref):
    o_ref[...] = jnp.dot(x_ref[...], w_ref[...], preferred_element_type=jnp.float32)


def _mm(x, w):
    M, K = x.shape
    _, N = w.shape
    bm = M
    for cand in (2000, 1024, 1000, 512, 256):
        if M > cand and M % cand == 0:
            bm = cand
            break
    return pl.pallas_call(
        _mm_body,
        grid=(M // bm,),
        in_specs=[
            pl.BlockSpec((bm, K), lambda i: (i, 0)),
            pl.BlockSpec((K, N), lambda i: (0, 0)),
        ],
        out_specs=pl.BlockSpec((bm, N), lambda i: (i, 0)),
        out_shape=jax.ShapeDtypeStruct((M, N), jnp.float32),
    )(x, w)


def _conv_as_matmul_weight(conv_w):
    rows, cols = [], []
    for f in range(NFILT):
        for i in range(2 * K_W - KER + 1):
            for j in range(K_H - KER + 1):
                for di in range(KER):
                    for dj in range(KER):
                        rows.append((i + di) * K_H + (j + dj))
                        cols.append(f * 100 + i * 10 + j)
    vals = jnp.broadcast_to(
        conv_w[:, 0, :, :].reshape(NFILT, 1, 1, KER, KER),
        (NFILT, 10, 10, KER, KER),
    ).reshape(-1)
    w2 = jnp.zeros((2 * K_W * K_H, FLAT_SZ), jnp.float32)
    return w2.at[np.array(rows), np.array(cols)].add(vals)


def _bn_rows(x, g, b):
    m = x.mean(0)
    v = x.var(0)
    return (x - m) / jnp.sqrt(v + 1e-5) * g + b


def _gat_layer(x, ef, et, src0, dst0, a_heads, a2_heads, in_dim, concat):
    nh = len(a_heads)
    hd = a_heads[0].shape[0]
    wall_cols = []
    for a, a2 in zip(a_heads, a2_heads):
        a_s = a[:, :in_dim]
        a_d = a[:, in_dim:2 * in_dim]
        wall_cols += [a_s.T, a_d.T, (a_s.T @ a2)[:, None], (a_d.T @ a2)[:, None]]
    wall = jnp.concatenate(wall_cols, axis=1)
    pre = _mm(x, wall)
    rall_cols = []
    for a, a2 in zip(a_heads, a2_heads):
        a_r = a[:, 2 * in_dim:]
        rall_cols += [a_r.T, (a_r.T @ a2)[:, None]]
    rall = jnp.concatenate(rall_cols, axis=1)
    rpre = _mm(ef, rall)

    outs = []
    zmins = []
    per = 2 * hd + 2
    rper = hd + 1
    head_data = []
    for hi in range(nh):
        S = pre[:, hi * per: hi * per + hd]
        D = pre[:, hi * per + hd: hi * per + 2 * hd]
        ps = pre[:, hi * per + 2 * hd]
        pd = pre[:, hi * per + 2 * hd + 1]
        Rr = rpre[:, hi * rper: hi * rper + hd]
        pr = rpre[:, hi * rper + hd]
        ps_new, pd_new = ps[NUM_ENT:], pd[NUM_ENT:]
        z1 = ps_new + pd[src0] + pr[et]
        z2 = ps[dst0] + pd_new + pr[et]
        z3 = ps_new[E // 2:] + pd_new[: E // 2] + pr[et[: E // 2]]
        zmins.append(jnp.minimum(jnp.minimum(z1.min(), z2.min()), z3.min()))
        head_data.append((S, D, Rr, z1, z2, z3))
    zmin = jnp.min(jnp.stack(zmins))
    maxp = -jax.nn.leaky_relu(zmin, ALPHA)
    for hi in range(nh):
        S, D, Rr, z1, z2, z3 = head_data[hi]
        w1 = jnp.exp(-jax.nn.leaky_relu(z1, ALPHA) - maxp)
        w2 = jnp.exp(-jax.nn.leaky_relu(z2, ALPHA) - maxp)
        w3 = jnp.exp(-jax.nn.leaky_relu(z3, ALPHA) - maxp)
        Ret = Rr[et]
        D_new = D[NUM_ENT:]
        den_new = w1.at[E // 2:].add(w3)
        num_new = w1[:, None] * (D[src0] + Ret)
        num_new = num_new.at[E // 2:].add(w3[:, None] * (D_new[: E // 2] + Ret[: E // 2]))
        num_new = num_new + S[NUM_ENT:] * den_new[:, None]
        h_new = num_new / (den_new[:, None] + 1e-16)
        den_ent = jax.ops.segment_sum(w2, dst0, num_segments=NUM_ENT)
        acc_ent = jax.ops.segment_sum(w2[:, None] * (D_new + Ret), dst0, num_segments=NUM_ENT)
        h_ent = (S[:NUM_ENT] * den_ent[:, None] + acc_ent) / (den_ent[:, None] + 1e-16)
        outs.append(jnp.concatenate([h_ent, h_new], axis=0))
    h = jnp.concatenate(outs, axis=1)
    return jax.nn.elu(h) if concat else h


@jax.jit
def _run(params, sub, rel, edge_index, edge_type):
    src0, dst0 = edge_index[0], edge_index[1]
    et = edge_type
    x = params["x"]
    ef = params["edge_feature"]
    h = _gat_layer(x, ef, et, src0, dst0,
                   [params["att1_a"][i] for i in range(HEADS)],
                   [params["att1_a2"][i] for i in range(HEADS)], INIT_DIM, True)
    h = _gat_layer(h, ef, et, src0, dst0,
                   [params["att2_a"][i] for i in range(HEADS)],
                   [params["att2_a2"][i] for i in range(HEADS)], EMBED_DIM, True)
    h = _gat_layer(h, ef, et, src0, dst0,
                   [params["out_a"]], [params["out_a2"]], EMBED_DIM, False)
    h = jax.nn.elu(h)

    xn = h[NUM_ENT:]
    counts = jax.ops.segment_sum(jnp.ones((E,), jnp.float32), et, num_segments=2 * NUM_REL)
    sums = jax.ops.segment_sum(xn, et, num_segments=2 * NUM_REL)
    edge_features = sums / jnp.clip(counts, 1.0)[:, None]
    edge_features = _mm(edge_features, params["W"])
    h = h + _mm(params["x"], params["W_entities"])
    h = _bn_rows(h, params["bn_g"], params["bn_b"])

    sub_emb = h[sub]
    rel_emb = edge_features[rel]
    stk = jnp.concatenate([sub_emb[:, None, :], rel_emb[:, None, :]], axis=1)
    img = jnp.transpose(stk, (0, 2, 1)).reshape(BATCH, 2 * K_W * K_H)
    m0 = img.mean()
    v0 = img.var()
    img = (img - m0) / jnp.sqrt(v0 + 1e-5) * params["bn0_g"][0] + params["bn0_b"][0]
    w2 = _conv_as_matmul_weight(params["conv_w"])
    c = _mm(img, w2)
    cr = c.reshape(BATCH, NFILT, 100)
    m1 = cr.mean((0, 2))
    v1 = cr.var((0, 2))
    cr = (cr - m1[None, :, None]) / jnp.sqrt(v1[None, :, None] + 1e-5)
    cr = cr * params["bn1_g"][None, :, None] + params["bn1_b"][None, :, None]
    c = jax.nn.relu(cr).reshape(BATCH, FLAT_SZ)
    c = _mm(c, params["fc_w"].T) + params["fc_b"]
    c = _bn_rows(c, params["bn2_g"], params["bn2_b"])
    c = jax.nn.relu(c)
    logits = _mm(c, h[:NUM_ENT].T) + params["bias1"][None, :]
    return jax.nn.sigmoid(logits)


def kernel(params, sub, rel, edge_index, edge_type):
    return _run(params, sub, rel, edge_index, edge_type)


# speedup vs baseline: 1.8836x; 1.0132x over previous
import functools
import numpy as np
import jax
import jax.numpy as jnp
from jax.experimental import pallas as pl

NUM_ENT = 10000
NUM_REL = 500
E = 160000
INIT_DIM = 128
EMBED_DIM = 128
HEADS = 2
HIDDEN = EMBED_DIM // HEADS
K_W, K_H = 8, 16
KER = 7
NFILT = 96
FLAT_SZ = (2 * K_W - KER + 1) * (K_H - KER + 1) * NFILT
BATCH = 1024
N_TOTAL = NUM_ENT + E
ALPHA = 0.2


def _mm_body(x_ref, w_ref, o_ref):
    o_ref[...] = jnp.dot(x_ref[...], w_ref[...], preferred_element_type=jnp.float32)


def _mm(x, w):
    M, K = x.shape
    _, N = w.shape
    bm = M
    for cand in (2000, 1024, 1000, 512, 256):
        if M > cand and M % cand == 0:
            bm = cand
            break
    return pl.pallas_call(
        _mm_body,
        grid=(M // bm,),
        in_specs=[
            pl.BlockSpec((bm, K), lambda i: (i, 0)),
            pl.BlockSpec((K, N), lambda i: (0, 0)),
        ],
        out_specs=pl.BlockSpec((bm, N), lambda i: (i, 0)),
        out_shape=jax.ShapeDtypeStruct((M, N), jnp.float32),
    )(x, w)


def _conv_im2col(img2d, conv_w):
    cols = [img2d[:, di:di + 10, dj:dj + 10].reshape(BATCH, 100)
            for di in range(KER) for dj in range(KER)]
    patches = jnp.stack(cols, axis=-1).reshape(BATCH * 100, KER * KER)
    wf = conv_w[:, 0].reshape(NFILT, KER * KER).T
    out = _mm(patches, wf)
    return jnp.transpose(out.reshape(BATCH, 100, NFILT), (0, 2, 1))


def _bn_rows(x, g, b):
    m = x.mean(0)
    v = x.var(0)
    return (x - m) / jnp.sqrt(v + 1e-5) * g + b


def _gat_layer(x, ef, et, src0, dst0, a_heads, a2_heads, in_dim, concat):
    nh = len(a_heads)
    hd = a_heads[0].shape[0]
    wall_cols = []
    for a, a2 in zip(a_heads, a2_heads):
        a_s = a[:, :in_dim]
        a_d = a[:, in_dim:2 * in_dim]
        wall_cols += [a_s.T, a_d.T, (a_s.T @ a2)[:, None], (a_d.T @ a2)[:, None]]
    wall = jnp.concatenate(wall_cols, axis=1)
    pre = _mm(x, wall)
    rall_cols = []
    for a, a2 in zip(a_heads, a2_heads):
        a_r = a[:, 2 * in_dim:]
        rall_cols += [a_r.T, (a_r.T @ a2)[:, None]]
    rall = jnp.concatenate(rall_cols, axis=1)
    rpre = _mm(ef, rall)

    outs = []
    zmins = []
    per = 2 * hd + 2
    rper = hd + 1
    head_data = []
    for hi in range(nh):
        S = pre[:, hi * per: hi * per + hd]
        D = pre[:, hi * per + hd: hi * per + 2 * hd]
        ps = pre[:, hi * per + 2 * hd]
        pd = pre[:, hi * per + 2 * hd + 1]
        Rr = rpre[:, hi * rper: hi * rper + hd]
        pr = rpre[:, hi * rper + hd]
        ps_new, pd_new = ps[NUM_ENT:], pd[NUM_ENT:]
        z1 = ps_new + pd[src0] + pr[et]
        z2 = ps[dst0] + pd_new + pr[et]
        z3 = ps_new[E // 2:] + pd_new[: E // 2] + pr[et[: E // 2]]
        zmins.append(jnp.minimum(jnp.minimum(z1.min(), z2.min()), z3.min()))
        head_data.append((S, D, Rr, z1, z2, z3))
    zmin = jnp.min(jnp.stack(zmins))
    maxp = -jax.nn.leaky_relu(zmin, ALPHA)
    for hi in range(nh):
        S, D, Rr, z1, z2, z3 = head_data[hi]
        w1 = jnp.exp(-jax.nn.leaky_relu(z1, ALPHA) - maxp)
        w2 = jnp.exp(-jax.nn.leaky_relu(z2, ALPHA) - maxp)
        w3 = jnp.exp(-jax.nn.leaky_relu(z3, ALPHA) - maxp)
        Ret = Rr[et]
        D_new = D[NUM_ENT:]
        den_new = jnp.concatenate([w1[: E // 2], w1[E // 2:] + w3])
        num1 = w1[:, None] * (D[src0] + Ret)
        num_hi = num1[E // 2:] + w3[:, None] * (D_new[: E // 2] + Ret[: E // 2])
        num_new = jnp.concatenate([num1[: E // 2], num_hi], axis=0)
        num_new = num_new + S[NUM_ENT:] * den_new[:, None]
        h_new = num_new / (den_new[:, None] + 1e-16)
        den_ent = jax.ops.segment_sum(w2, dst0, num_segments=NUM_ENT)
        acc_ent = jax.ops.segment_sum(w2[:, None] * (D_new + Ret), dst0, num_segments=NUM_ENT)
        h_ent = (S[:NUM_ENT] * den_ent[:, None] + acc_ent) / (den_ent[:, None] + 1e-16)
        outs.append(jnp.concatenate([h_ent, h_new], axis=0))
    h = jnp.concatenate(outs, axis=1)
    return jax.nn.elu(h) if concat else h


@jax.jit
def _run(params, sub, rel, edge_index, edge_type):
    src0, dst0 = edge_index[0], edge_index[1]
    et = edge_type
    x = params["x"]
    ef = params["edge_feature"]
    h = _gat_layer(x, ef, et, src0, dst0,
                   [params["att1_a"][i] for i in range(HEADS)],
                   [params["att1_a2"][i] for i in range(HEADS)], INIT_DIM, True)
    h = _gat_layer(h, ef, et, src0, dst0,
                   [params["att2_a"][i] for i in range(HEADS)],
                   [params["att2_a2"][i] for i in range(HEADS)], EMBED_DIM, True)
    h = _gat_layer(h, ef, et, src0, dst0,
                   [params["out_a"]], [params["out_a2"]], EMBED_DIM, False)
    h = jax.nn.elu(h)

    xn = h[NUM_ENT:]
    counts = jax.ops.segment_sum(jnp.ones((E,), jnp.float32), et, num_segments=2 * NUM_REL)
    sums = jax.ops.segment_sum(xn, et, num_segments=2 * NUM_REL)
    edge_features = sums / jnp.clip(counts, 1.0)[:, None]
    edge_features = _mm(edge_features, params["W"])
    h = h + _mm(params["x"], params["W_entities"])
    h = _bn_rows(h, params["bn_g"], params["bn_b"])

    sub_emb = h[sub]
    rel_emb = edge_features[rel]
    stk = jnp.concatenate([sub_emb[:, None, :], rel_emb[:, None, :]], axis=1)
    img = jnp.transpose(stk, (0, 2, 1)).reshape(BATCH, 2 * K_W * K_H)
    m0 = img.mean()
    v0 = img.var()
    img = (img - m0) / jnp.sqrt(v0 + 1e-5) * params["bn0_g"][0] + params["bn0_b"][0]
    cr = _conv_im2col(img.reshape(BATCH, 2 * K_W, K_H), params["conv_w"])
    m1 = cr.mean((0, 2))
    v1 = cr.var((0, 2))
    cr = (cr - m1[None, :, None]) / jnp.sqrt(v1[None, :, None] + 1e-5)
    cr = cr * params["bn1_g"][None, :, None] + params["bn1_b"][None, :, None]
    c = jax.nn.relu(cr).reshape(BATCH, FLAT_SZ)
    c = _mm(c, params["fc_w"].T) + params["fc_b"]
    c = _bn_rows(c, params["bn2_g"], params["bn2_b"])
    c = jax.nn.relu(c)
    logits = _mm(c, h[:NUM_ENT].T) + params["bias1"][None, :]
    return jax.nn.sigmoid(logits)


def kernel(params, sub, rel, edge_index, edge_type):
    return _run(params, sub, rel, edge_index, edge_type)


# speedup vs baseline: 2.2751x; 1.2078x over previous
import functools
import numpy as np
import jax
import jax.numpy as jnp
from jax import lax
from jax.experimental import pallas as pl
from jax.experimental.pallas import tpu as pltpu
from jax.experimental.pallas import tpu_sc as plsc

NUM_ENT = 10000
NUM_REL = 500
E = 160000
INIT_DIM = 128
EMBED_DIM = 128
HEADS = 2
HIDDEN = EMBED_DIM // HEADS
K_W, K_H = 8, 16
KER = 7
NFILT = 96
FLAT_SZ = (2 * K_W - KER + 1) * (K_H - KER + 1) * NFILT
BATCH = 1024
N_TOTAL = NUM_ENT + E
ALPHA = 0.2


_CH = 128
_NW = 32


def _sc_gather(table, idx):
    V, D = table.shape
    B = idx.shape[0]
    nch = B // _CH
    per_w = (nch + _NW - 1) // _NW
    mesh = plsc.VectorSubcoreMesh(core_axis_name="c", subcore_axis_name="s")

    @functools.partial(
        pl.kernel,
        mesh=mesh,
        out_type=jax.ShapeDtypeStruct((B, D), jnp.float32),
        compiler_params=pltpu.CompilerParams(use_tc_tiling_on_sc=False),
        scratch_types=[
            pltpu.VMEM((_CH,), jnp.int32),
            pltpu.VMEM((_CH, D), jnp.float32),
            pltpu.SemaphoreType.DMA,
        ],
    )
    def k(table_hbm, idx_hbm, out_hbm, idx_v, rows_v, sem):
        wid = lax.axis_index("s") * 2 + lax.axis_index("c")

        def body(i, carry):
            c = i * _NW + wid

            @pl.when(c < nch)
            def _():
                base = c * _CH
                pltpu.sync_copy(idx_hbm.at[pl.ds(base, _CH)], idx_v)
                pltpu.async_copy(table_hbm.at[idx_v], rows_v, sem).wait()
                pltpu.sync_copy(rows_v, out_hbm.at[pl.ds(base, _CH)])

            return carry

        lax.fori_loop(0, per_w, body, 0)

    return k(table, idx)


def _sc_scatter_add(val, idx, nseg):
    B, D = val.shape
    nch = B // _CH
    per_w = (nch + _NW - 1) // _NW
    rps = nseg // 16
    mesh = plsc.VectorSubcoreMesh(core_axis_name="c", subcore_axis_name="s")

    @functools.partial(
        pl.kernel,
        mesh=mesh,
        out_type=jax.ShapeDtypeStruct((2, nseg, D), jnp.float32),
        compiler_params=pltpu.CompilerParams(use_tc_tiling_on_sc=False),
        scratch_types=[
            pltpu.VMEM((_CH,), jnp.int32),
            pltpu.VMEM((_CH, D), jnp.float32),
            pltpu.VMEM_SHARED((nseg, D), jnp.float32),
            pltpu.SemaphoreType.DMA,
        ],
    )
    def k(val_hbm, idx_hbm, zeros_hbm, out_hbm, idx_v, val_v, acc_sh, sem):
        cid = lax.axis_index("c")
        sid = lax.axis_index("s")
        wid = sid * 2 + cid
        r0 = sid * rps
        pltpu.sync_copy(zeros_hbm.at[pl.ds(r0, rps)], acc_sh.at[pl.ds(r0, rps)])
        plsc.subcore_barrier()

        def body(i, carry):
            c = i * _NW + wid

            @pl.when(c < nch)
            def _():
                base = c * _CH
                pltpu.sync_copy(idx_hbm.at[pl.ds(base, _CH)], idx_v)
                pltpu.sync_copy(val_hbm.at[pl.ds(base, _CH)], val_v)
                pltpu.sync_copy(val_v, acc_sh.at[idx_v], add=True)

            return carry

        lax.fori_loop(0, per_w, body, 0)
        plsc.subcore_barrier()
        done = 0
        while done < rps:
            rows = min(_CH, rps - done)
            pltpu.sync_copy(acc_sh.at[pl.ds(r0 + done, rows)], val_v.at[pl.ds(0, rows)])
            pltpu.sync_copy(val_v.at[pl.ds(0, rows)], out_hbm.at[cid, pl.ds(r0 + done, rows)])
            done += rows

    zeros = jnp.zeros((nseg, D), jnp.float32)
    return k(val, idx, zeros)


def _mm_body(x_ref, w_ref, o_ref):
    o_ref[...] = jnp.dot(x_ref[...], w_ref[...], preferred_element_type=jnp.float32)


def _mm(x, w):
    M, K = x.shape
    _, N = w.shape
    bm = M
    for cand in (2000, 1024, 1000, 512, 256):
        if M > cand and M % cand == 0:
            bm = cand
            break
    return pl.pallas_call(
        _mm_body,
        grid=(M // bm,),
        in_specs=[
            pl.BlockSpec((bm, K), lambda i: (i, 0)),
            pl.BlockSpec((K, N), lambda i: (0, 0)),
        ],
        out_specs=pl.BlockSpec((bm, N), lambda i: (i, 0)),
        out_shape=jax.ShapeDtypeStruct((M, N), jnp.float32),
    )(x, w)


def _conv_im2col(img2d, conv_w):
    cols = [img2d[:, di:di + 10, dj:dj + 10].reshape(BATCH, 100)
            for di in range(KER) for dj in range(KER)]
    patches = jnp.stack(cols, axis=-1).reshape(BATCH * 100, KER * KER)
    wf = conv_w[:, 0].reshape(NFILT, KER * KER).T
    out = _mm(patches, wf)
    return jnp.transpose(out.reshape(BATCH, 100, NFILT), (0, 2, 1))


def _bn_rows(x, g, b):
    m = x.mean(0)
    v = x.var(0)
    return (x - m) / jnp.sqrt(v + 1e-5) * g + b


def _gat_layer(x, ef, et, src0, dst0, a_heads, a2_heads, in_dim, concat):
    nh = len(a_heads)
    hd = a_heads[0].shape[0]
    wall_cols = []
    for a, a2 in zip(a_heads, a2_heads):
        a_s = a[:, :in_dim]
        a_d = a[:, in_dim:2 * in_dim]
        wall_cols += [a_s.T, a_d.T, (a_s.T @ a2)[:, None], (a_d.T @ a2)[:, None]]
    wall = jnp.concatenate(wall_cols, axis=1)
    pre = _mm(x, wall)
    rall_cols = []
    for a, a2 in zip(a_heads, a2_heads):
        a_r = a[:, 2 * in_dim:]
        rall_cols += [a_r.T, (a_r.T @ a2)[:, None]]
    rall = jnp.concatenate(rall_cols, axis=1)
    rpre = _mm(ef, rall)

    per = 2 * hd + 2
    rper = hd + 1
    nhd = nh * hd
    padc = jnp.zeros((NUM_ENT, 16 - nh), jnp.float32)
    ts = jnp.concatenate(
        [pre[:NUM_ENT, hi * per + hd: hi * per + 2 * hd] for hi in range(nh)]
        + [pre[:NUM_ENT, hi * per + 2 * hd + 1: hi * per + 2 * hd + 2] for hi in range(nh)]
        + [padc], axis=1)
    pste = jnp.concatenate(
        [pre[:NUM_ENT, hi * per + 2 * hd: hi * per + 2 * hd + 1] for hi in range(nh)]
        + [padc], axis=1)
    padr = jnp.zeros((2 * NUM_REL, 16 - nh), jnp.float32)
    rc = jnp.concatenate(
        [rpre[:, hi * rper: hi * rper + hd] for hi in range(nh)]
        + [rpre[:, hi * rper + hd: hi * rper + hd + 1] for hi in range(nh)]
        + [padr], axis=1)
    gsrc = _sc_gather(ts, src0)
    gdst = _sc_gather(pste, dst0)
    gret = _sc_gather(rc, et)

    zmins = []
    head_data = []
    for hi in range(nh):
        ps_new = pre[NUM_ENT:, hi * per + 2 * hd]
        pd_new = pre[NUM_ENT:, hi * per + 2 * hd + 1]
        pd_src = gsrc[:, nhd + hi]
        ps_dst = gdst[:, hi]
        pr_et = gret[:, nhd + hi]
        z1 = ps_new + pd_src + pr_et
        z2 = ps_dst + pd_new + pr_et
        z3 = ps_new[E // 2:] + pd_new[: E // 2] + pr_et[: E // 2]
        zmins.append(jnp.minimum(jnp.minimum(z1.min(), z2.min()), z3.min()))
        head_data.append((z1, z2, z3))
    zmin = jnp.min(jnp.stack(zmins))
    maxp = -jax.nn.leaky_relu(zmin, ALPHA)

    h_news = []
    val_cols = []
    w2_cols = []
    w2s = []
    Ss = []
    for hi in range(nh):
        z1, z2, z3 = head_data[hi]
        S = pre[:, hi * per: hi * per + hd]
        D_new = pre[NUM_ENT:, hi * per + hd: hi * per + 2 * hd]
        Dsrc = gsrc[:, hi * hd: (hi + 1) * hd]
        Ret = gret[:, hi * hd: (hi + 1) * hd]
        w1 = jnp.exp(-jax.nn.leaky_relu(z1, ALPHA) - maxp)
        w2 = jnp.exp(-jax.nn.leaky_relu(z2, ALPHA) - maxp)
        w3 = jnp.exp(-jax.nn.leaky_relu(z3, ALPHA) - maxp)
        den_new = jnp.concatenate([w1[: E // 2], w1[E // 2:] + w3])
        num1 = w1[:, None] * (Dsrc + Ret)
        num_hi = num1[E // 2:] + w3[:, None] * (D_new[: E // 2] + Ret[: E // 2])
        num_new = jnp.concatenate([num1[: E // 2], num_hi], axis=0)
        num_new = num_new + S[NUM_ENT:] * den_new[:, None]
        h_news.append(num_new / (den_new[:, None] + 1e-16))
        val_cols.append(w2[:, None] * (D_new + Ret))
        w2_cols.append(w2[:, None])
        w2s.append(w2)
        Ss.append(S)
    val = jnp.concatenate(
        val_cols + w2_cols + [jnp.zeros((E, 16 - nh), jnp.float32)], axis=1)
    acc2 = _sc_scatter_add(val, dst0, NUM_ENT)
    acc = acc2[0] + acc2[1]
    outs = []
    for hi in range(nh):
        acc_ent = acc[:, hi * hd: (hi + 1) * hd]
        den_ent = acc[:, nhd + hi]
        h_ent = (Ss[hi][:NUM_ENT] * den_ent[:, None] + acc_ent) / (den_ent[:, None] + 1e-16)
        outs.append(jnp.concatenate([h_ent, h_news[hi]], axis=0))
    h = jnp.concatenate(outs, axis=1)
    return jax.nn.elu(h) if concat else h


@jax.jit
def _run(params, sub, rel, edge_index, edge_type):
    src0, dst0 = edge_index[0], edge_index[1]
    et = edge_type
    x = params["x"]
    ef = params["edge_feature"]
    h = _gat_layer(x, ef, et, src0, dst0,
                   [params["att1_a"][i] for i in range(HEADS)],
                   [params["att1_a2"][i] for i in range(HEADS)], INIT_DIM, True)
    h = _gat_layer(h, ef, et, src0, dst0,
                   [params["att2_a"][i] for i in range(HEADS)],
                   [params["att2_a2"][i] for i in range(HEADS)], EMBED_DIM, True)
    h = _gat_layer(h, ef, et, src0, dst0,
                   [params["out_a"]], [params["out_a2"]], EMBED_DIM, False)
    h = jax.nn.elu(h)

    xn = h[NUM_ENT:]
    valf = jnp.concatenate([xn, jnp.ones((E, 16), jnp.float32)], axis=1)
    accf2 = _sc_scatter_add(valf, et, 1008)
    accf = accf2[0] + accf2[1]
    sums = accf[: 2 * NUM_REL, :EMBED_DIM]
    counts = accf[: 2 * NUM_REL, EMBED_DIM]
    edge_features = sums / jnp.clip(counts, 1.0)[:, None]
    edge_features = _mm(edge_features, params["W"])
    h = h + _mm(params["x"], params["W_entities"])
    h = _bn_rows(h, params["bn_g"], params["bn_b"])

    sub_emb = _sc_gather(h, sub)
    rel_emb = _sc_gather(edge_features, rel)
    stk = jnp.concatenate([sub_emb[:, None, :], rel_emb[:, None, :]], axis=1)
    img = jnp.transpose(stk, (0, 2, 1)).reshape(BATCH, 2 * K_W * K_H)
    m0 = img.mean()
    v0 = img.var()
    img = (img - m0) / jnp.sqrt(v0 + 1e-5) * params["bn0_g"][0] + params["bn0_b"][0]
    cr = _conv_im2col(img.reshape(BATCH, 2 * K_W, K_H), params["conv_w"])
    m1 = cr.mean((0, 2))
    v1 = cr.var((0, 2))
    cr = (cr - m1[None, :, None]) / jnp.sqrt(v1[None, :, None] + 1e-5)
    cr = cr * params["bn1_g"][None, :, None] + params["bn1_b"][None, :, None]
    c = jax.nn.relu(cr).reshape(BATCH, FLAT_SZ)
    c = _mm(c, params["fc_w"].T) + params["fc_b"]
    c = _bn_rows(c, params["bn2_g"], params["bn2_b"])
    c = jax.nn.relu(c)
    logits = _mm(c, h[:NUM_ENT].T) + params["bias1"][None, :]
    return jax.nn.sigmoid(logits)


def kernel(params, sub, rel, edge_index, edge_type):
    return _run(params, sub, rel, edge_index, edge_type)
